# baseline (speedup 1.0000x reference)
import functools

import jax
import jax.numpy as jnp
from jax import lax
from jax.experimental import pallas as pl
from jax.experimental.pallas import tpu as pltpu
from jax.experimental.pallas import tpu_sc as plsc

_N = 10000
_NP = 10240
_E = 320000
_D = 128
_NC = 2
_NS = 16
_NW = _NC * _NS
_EPW = _E // _NW
_K = 80
_CH = _EPW // _K
_U = 5
_UPW = _CH
_RPT = _NP // _NS
_BN = 2048
_GN = _NP // _BN


def _lrelu(v):
    return jnp.where(v >= 0, v, 0.2 * v)


def _prep_body(x_ref, w_ref, ats_ref, atd_ref, ew_ref, we1_ref, ate1_ref,
               we2_ref, ate2_ref,
               xp_ref, as_ref, ad_ref, ae1_ref, ae2_ref, m1_ref, m2_ref):
    i = pl.program_id(0)
    xp = jnp.dot(x_ref[...], w_ref[...], preferred_element_type=jnp.float32)
    xp_ref[...] = xp
    as_ref[...] = jnp.sum(xp * ats_ref[...], axis=-1, keepdims=True)
    ad_ref[...] = jnp.sum(xp * atd_ref[...], axis=-1, keepdims=True)
    c1 = jnp.sum(we1_ref[...] * ate1_ref[...])
    c2 = jnp.sum(we2_ref[...] * ate2_ref[...])
    ew = ew_ref[...]
    ae1_ref[...] = ew * c1
    ae2_ref[...] = ew * c2
    @pl.when(i == 0)
    def _():
        mean = (jnp.sum(ew) * (1.0 / _E)).reshape(1, 1)
        m1_ref[...] = mean * c1
        m2_ref[...] = mean * c2


def _tc_prep(x, W, ats, atd, ew2d, we1, ate1, we2, ate2):
    eb = _E // _D
    return pl.pallas_call(
        _prep_body,
        grid=(_GN,),
        in_specs=[
            pl.BlockSpec((_BN, _D), lambda i: (i, 0)),
            pl.BlockSpec((_D, _D), lambda i: (0, 0)),
            pl.BlockSpec((1, _D), lambda i: (0, 0)),
            pl.BlockSpec((1, _D), lambda i: (0, 0)),
            pl.BlockSpec((eb, _D), lambda i: (0, 0)),
            pl.BlockSpec((1, _D), lambda i: (0, 0)),
            pl.BlockSpec((1, _D), lambda i: (0, 0)),
            pl.BlockSpec((1, _D), lambda i: (0, 0)),
            pl.BlockSpec((1, _D), lambda i: (0, 0)),
        ],
        out_specs=[
            pl.BlockSpec((_BN, _D), lambda i: (i, 0)),
            pl.BlockSpec((_BN, 1), lambda i: (i, 0)),
            pl.BlockSpec((_BN, 1), lambda i: (i, 0)),
            pl.BlockSpec((eb, _D), lambda i: (0, 0)),
            pl.BlockSpec((eb, _D), lambda i: (0, 0)),
            pl.BlockSpec((1, 1), lambda i: (0, 0)),
            pl.BlockSpec((1, 1), lambda i: (0, 0)),
        ],
        out_shape=[
            jax.ShapeDtypeStruct((_NP, _D), jnp.float32),
            jax.ShapeDtypeStruct((_NP, 1), jnp.float32),
            jax.ShapeDtypeStruct((_NP, 1), jnp.float32),
            jax.ShapeDtypeStruct((_E // _D, _D), jnp.float32),
            jax.ShapeDtypeStruct((_E // _D, _D), jnp.float32),
            jax.ShapeDtypeStruct((1, 1), jnp.float32),
            jax.ShapeDtypeStruct((1, 1), jnp.float32),
        ],
    )(x, W, ats, atd, ew2d, we1, ate1, we2, ate2)


def _sc_body(src_h, dst_h, ae_h, xp_h, as_h, ad_h, z_h, zd_h,
             out_h, den_h,
             asv, adv, srcv, dstv, aev, pv, rows, outsp, densp,
             semi, semg, sems):
    c = lax.axis_index("c")
    s = lax.axis_index("s")
    pltpu.sync_copy(z_h.at[pl.ds(s * _RPT, _RPT)], outsp.at[pl.ds(s * _RPT, _RPT)])
    pltpu.sync_copy(zd_h.at[pl.ds(s * _RPT, _RPT)], densp.at[pl.ds(s * _RPT, _RPT)])
    pltpu.sync_copy(as_h, asv)
    pltpu.sync_copy(ad_h, adv)
    plsc.subcore_barrier()

    base_e = (c * _NS + s) * _EPW

    def idx_copies(g):
        slot = lax.rem(g, 3)
        off = pl.multiple_of(base_e + g * _K, 8)
        return (
            pltpu.make_async_copy(src_h.at[pl.ds(off, _K)], srcv.at[slot],
                                  semi.at[slot]),
            pltpu.make_async_copy(dst_h.at[pl.ds(off, _K)], dstv.at[slot],
                                  semi.at[slot]),
            pltpu.make_async_copy(ae_h.at[pl.ds(off, _K)], aev.at[slot],
                                  semi.at[slot]),
        )

    def gather_copy(g):
        b = lax.rem(g, 2)
        slot = lax.rem(g, 3)
        return pltpu.make_async_copy(xp_h.at[srcv.at[slot]], rows.at[b],
                                     semg.at[b])

    def scatter_copies(g):
        b = lax.rem(g, 2)
        slot = lax.rem(g, 3)
        return (
            pltpu.make_async_copy(rows.at[b], outsp.at[dstv.at[slot]],
                                  sems.at[b]),
            pltpu.make_async_copy(pv.at[b], densp.at[dstv.at[slot]],
                                  sems.at[b]),
        )

    for cp in idx_copies(0):
        cp.start()
    for cp in idx_copies(0):
        cp.wait()
    gather_copy(0).start()
    for cp in idx_copies(1):
        cp.start()

    def chunk(g, carry):
        b = lax.rem(g, 2)
        slot = lax.rem(g, 3)
        for i in range(_K // 16):
            sl = pl.ds(i * 16, 16)
            av = plsc.load_gather(asv, [srcv[slot, sl]])
            bv = plsc.load_gather(adv, [dstv[slot, sl]])
            al = _lrelu(av + bv + aev[slot, sl])
            pv[b, sl] = jnp.exp(al)
        gather_copy(g).wait()

        def scale_grp(grp, cc):
            pvv = pv[b, pl.ds(grp * 16, 16)]
            row0 = grp * 16
            for r in range(16):
                p = pvv[r]
                for j in range(_D // 16):
                    sl2 = pl.ds(j * 16, 16)
                    rows[b, row0 + r, sl2] = rows[b, row0 + r, sl2] * p
            return cc

        lax.fori_loop(0, _K // 16, scale_grp, 0)

        @pl.when(g >= 1)
        def _():
            for cp in scatter_copies(g - 1):
                cp.wait()

        @pl.when(g + 1 < _CH)
        def _():
            for cp in idx_copies(g + 1):
                cp.wait()
            gather_copy(g + 1).start()

        @pl.when(g + 2 < _CH)
        def _():
            for cp in idx_copies(g + 2):
                cp.start()

        for cp in scatter_copies(g):
            cp.start(add=True)
        return carry

    lax.fori_loop(0, _CH, chunk, 0)
    for cp in scatter_copies(_CH - 1):
        cp.wait()
    plsc.subcore_barrier()
    pltpu.sync_copy(outsp.at[pl.ds(s * _RPT, _RPT)],
                    out_h.at[pl.ds(c * _NP + s * _RPT, _RPT)])
    pltpu.sync_copy(densp.at[pl.ds(s * _RPT, _RPT)],
                    den_h.at[pl.ds(c * _NP + s * _RPT, _RPT)])


@functools.cache
def _sc_edge():
    return pl.kernel(
        _sc_body,
        mesh=plsc.VectorSubcoreMesh(core_axis_name="c", subcore_axis_name="s"),
        compiler_params=pltpu.CompilerParams(needs_layout_passes=False),
        out_type=[
            jax.ShapeDtypeStruct((_NC * _NP, _D), jnp.float32),
            jax.ShapeDtypeStruct((_NC * _NP,), jnp.float32),
        ],
        scratch_types=[
            pltpu.VMEM((_NP,), jnp.float32),
            pltpu.VMEM((_NP,), jnp.float32),
            pltpu.VMEM((3, _K), jnp.int32),
            pltpu.VMEM((3, _K), jnp.int32),
            pltpu.VMEM((3, _K), jnp.float32),
            pltpu.VMEM((2, _K), jnp.float32),
            pltpu.VMEM((2, _K, _D), jnp.float32),
            pltpu.VMEM_SHARED((_NP, _D), jnp.float32),
            pltpu.VMEM_SHARED((_NP,), jnp.float32),
            pltpu.SemaphoreType.DMA((3,)),
            pltpu.SemaphoreType.DMA((2,)),
            pltpu.SemaphoreType.DMA((2,)),
        ],
    )


def _comb1_body(op_ref, d_ref, xp_ref, as_ref, ad_ref, m_ref, b_ref,
                w2_ref, ats_ref, atd_ref, xp2_ref, as2_ref, ad2_ref):
    ploop = jnp.exp(_lrelu(as_ref[...] + ad_ref[...] + m_ref[...]))
    num = op_ref[0] + op_ref[1] + ploop * xp_ref[...]
    den = d_ref[0] + d_ref[1] + ploop + 1e-16
    h = num / den + b_ref[...]
    h = jnp.where(h > 0, h, jnp.exp(jnp.minimum(h, 0.0)) - 1.0)
    xp2 = jnp.dot(h, w2_ref[...], preferred_element_type=jnp.float32)
    xp2_ref[...] = xp2
    as2_ref[...] = jnp.sum(xp2 * ats_ref[...], axis=-1, keepdims=True)
    ad2_ref[...] = jnp.sum(xp2 * atd_ref[...], axis=-1, keepdims=True)


def _tc_comb1(op, dp, xp, a_s, a_d, m, b, W2, ats2, atd2):
    return pl.pallas_call(
        _comb1_body,
        grid=(_GN,),
        in_specs=[
            pl.BlockSpec((_NC, _BN, _D), lambda i: (0, i, 0)),
            pl.BlockSpec((_NC, _BN, 1), lambda i: (0, i, 0)),
            pl.BlockSpec((_BN, _D), lambda i: (i, 0)),
            pl.BlockSpec((_BN, 1), lambda i: (i, 0)),
            pl.BlockSpec((_BN, 1), lambda i: (i, 0)),
            pl.BlockSpec((1, 1), lambda i: (0, 0)),
            pl.BlockSpec((1, _D), lambda i: (0, 0)),
            pl.BlockSpec((_D, _D), lambda i: (0, 0)),
            pl.BlockSpec((1, _D), lambda i: (0, 0)),
            pl.BlockSpec((1, _D), lambda i: (0, 0)),
        ],
        out_specs=[
            pl.BlockSpec((_BN, _D), lambda i: (i, 0)),
            pl.BlockSpec((_BN, 1), lambda i: (i, 0)),
            pl.BlockSpec((_BN, 1), lambda i: (i, 0)),
        ],
        out_shape=[
            jax.ShapeDtypeStruct((_NP, _D), jnp.float32),
            jax.ShapeDtypeStruct((_NP, 1), jnp.float32),
            jax.ShapeDtypeStruct((_NP, 1), jnp.float32),
        ],
    )(op, dp, xp, a_s, a_d, m, b, W2, ats2, atd2)


def _comb2_body(op_ref, d_ref, xp_ref, as_ref, ad_ref, m_ref, b_ref, o_ref):
    ploop = jnp.exp(_lrelu(as_ref[...] + ad_ref[...] + m_ref[...]))
    num = op_ref[0] + op_ref[1] + ploop * xp_ref[...]
    den = d_ref[0] + d_ref[1] + ploop + 1e-16
    o_ref[...] = num / den + b_ref[...]


def _tc_comb2(op, dp, xp, a_s, a_d, m, b):
    return pl.pallas_call(
        _comb2_body,
        grid=(_GN,),
        in_specs=[
            pl.BlockSpec((_NC, _BN, _D), lambda i: (0, i, 0)),
            pl.BlockSpec((_NC, _BN, 1), lambda i: (0, i, 0)),
            pl.BlockSpec((_BN, _D), lambda i: (i, 0)),
            pl.BlockSpec((_BN, 1), lambda i: (i, 0)),
            pl.BlockSpec((_BN, 1), lambda i: (i, 0)),
            pl.BlockSpec((1, 1), lambda i: (0, 0)),
            pl.BlockSpec((1, _D), lambda i: (0, 0)),
        ],
        out_specs=pl.BlockSpec((_BN, _D), lambda i: (i, 0)),
        out_shape=jax.ShapeDtypeStruct((_NP, _D), jnp.float32),
    )(op, dp, xp, a_s, a_d, m, b)


def kernel(x, edge_index, edge_weight, W1, W_edge1, att_src1, att_dst1,
           att_edge1, bias1, W2, W_edge2, att_src2, att_dst2, att_edge2,
           bias2):
    xpad = jnp.pad(x, ((0, _NP - _N), (0, 0)))
    src1d = edge_index[0]
    dst1d = edge_index[1]
    ew2d = edge_weight.reshape(_E // _D, _D)
    ats1 = att_src1.reshape(1, _D)
    atd1 = att_dst1.reshape(1, _D)
    ate1 = att_edge1.reshape(1, _D)
    ats2 = att_src2.reshape(1, _D)
    atd2 = att_dst2.reshape(1, _D)
    ate2 = att_edge2.reshape(1, _D)
    b1 = bias1.reshape(1, _D)
    b2 = bias2.reshape(1, _D)
    zrow = jnp.zeros((_NP, _D), jnp.float32)
    zd = jnp.zeros((_NP,), jnp.float32)

    xp1, as1, ad1, ae1, ae2, m1, m2 = _tc_prep(
        xpad, W1, ats1, atd1, ew2d, W_edge1.reshape(1, _D), ate1,
        W_edge2.reshape(1, _D), ate2)
    op1, dp1 = _sc_edge()(src1d, dst1d, ae1.reshape(_E), xp1,
                          as1.reshape(_NP), ad1.reshape(_NP), zrow, zd)
    xp2, as2, ad2 = _tc_comb1(op1.reshape(_NC, _NP, _D),
                              dp1.reshape(_NC, _NP, 1),
                              xp1, as1, ad1, m1, b1, W2, ats2, atd2)
    op2, dp2 = _sc_edge()(src1d, dst1d, ae2.reshape(_E), xp2,
                          as2.reshape(_NP), ad2.reshape(_NP), zrow, zd)
    out = _tc_comb2(op2.reshape(_NC, _NP, _D), dp2.reshape(_NC, _NP, 1),
                    xp2, as2, ad2, m2, b2)
    return out[:_N]


# speedup vs baseline: 1.7916x; 1.7916x over previous
import functools

import jax
import jax.numpy as jnp
from jax import lax
from jax.experimental import pallas as pl
from jax.experimental.pallas import tpu as pltpu
from jax.experimental.pallas import tpu_sc as plsc

_N = 10000
_NP = 10240
_E = 320000
_D = 128
_NC = 2
_NS = 16
_NW = _NC * _NS
_EPW = _E // _NW
_K = 80
_CH = _EPW // _K
_U = 5
_UPW = _CH
_RPT = _NP // _NS
_BN = 2048
_GN = _NP // _BN


def _lrelu(v):
    return jnp.where(v >= 0, v, 0.2 * v)


def _prep_body(x_ref, w_ref, ats_ref, atd_ref, ew_ref, we1_ref, ate1_ref,
               we2_ref, ate2_ref,
               xp_ref, as_ref, ad_ref, ae1_ref, ae2_ref, m1_ref, m2_ref):
    i = pl.program_id(0)
    xp = jnp.dot(x_ref[...], w_ref[...], preferred_element_type=jnp.float32)
    xp_ref[...] = xp
    as_ref[...] = jnp.sum(xp * ats_ref[...], axis=-1, keepdims=True)
    ad_ref[...] = jnp.sum(xp * atd_ref[...], axis=-1, keepdims=True)
    c1 = jnp.sum(we1_ref[...] * ate1_ref[...])
    c2 = jnp.sum(we2_ref[...] * ate2_ref[...])
    ew = ew_ref[...]
    ae1_ref[...] = ew * c1
    ae2_ref[...] = ew * c2
    @pl.when(i == 0)
    def _():
        mean = (jnp.sum(ew) * (1.0 / _E)).reshape(1, 1)
        m1_ref[...] = mean * c1
        m2_ref[...] = mean * c2


def _tc_prep(x, W, ats, atd, ew2d, we1, ate1, we2, ate2):
    eb = _E // _D
    return pl.pallas_call(
        _prep_body,
        grid=(_GN,),
        in_specs=[
            pl.BlockSpec((_BN, _D), lambda i: (i, 0)),
            pl.BlockSpec((_D, _D), lambda i: (0, 0)),
            pl.BlockSpec((1, _D), lambda i: (0, 0)),
            pl.BlockSpec((1, _D), lambda i: (0, 0)),
            pl.BlockSpec((eb, _D), lambda i: (0, 0)),
            pl.BlockSpec((1, _D), lambda i: (0, 0)),
            pl.BlockSpec((1, _D), lambda i: (0, 0)),
            pl.BlockSpec((1, _D), lambda i: (0, 0)),
            pl.BlockSpec((1, _D), lambda i: (0, 0)),
        ],
        out_specs=[
            pl.BlockSpec((_BN, _D), lambda i: (i, 0)),
            pl.BlockSpec((_BN, 1), lambda i: (i, 0)),
            pl.BlockSpec((_BN, 1), lambda i: (i, 0)),
            pl.BlockSpec((eb, _D), lambda i: (0, 0)),
            pl.BlockSpec((eb, _D), lambda i: (0, 0)),
            pl.BlockSpec((1, 1), lambda i: (0, 0)),
            pl.BlockSpec((1, 1), lambda i: (0, 0)),
        ],
        out_shape=[
            jax.ShapeDtypeStruct((_NP, _D), jnp.float32),
            jax.ShapeDtypeStruct((_NP, 1), jnp.float32),
            jax.ShapeDtypeStruct((_NP, 1), jnp.float32),
            jax.ShapeDtypeStruct((_E // _D, _D), jnp.float32),
            jax.ShapeDtypeStruct((_E // _D, _D), jnp.float32),
            jax.ShapeDtypeStruct((1, 1), jnp.float32),
            jax.ShapeDtypeStruct((1, 1), jnp.float32),
        ],
    )(x, W, ats, atd, ew2d, we1, ate1, we2, ate2)


def _sc_body(src_h, dst_h, ae_h, xp_h, as_h, ad_h, z_h, zd_h,
             out_h, den_h,
             asv, adv, srcv, dstv, aev, pv, rows, outsp, densp,
             semi, semg, sems):
    c = lax.axis_index("c")
    s = lax.axis_index("s")
    pltpu.sync_copy(z_h.at[pl.ds(s * _RPT, _RPT)], outsp.at[pl.ds(s * _RPT, _RPT)])
    pltpu.sync_copy(zd_h.at[pl.ds(s * _RPT, _RPT)], densp.at[pl.ds(s * _RPT, _RPT)])
    pltpu.sync_copy(as_h, asv)
    pltpu.sync_copy(ad_h, adv)
    plsc.subcore_barrier()

    base_e = (c * _NS + s) * _EPW

    def idx_copies(g):
        slot = lax.rem(g, 3)
        off = pl.multiple_of(base_e + g * _K, 8)
        return (
            pltpu.make_async_copy(src_h.at[pl.ds(off, _K)], srcv.at[slot],
                                  semi.at[slot]),
            pltpu.make_async_copy(dst_h.at[pl.ds(off, _K)], dstv.at[slot],
                                  semi.at[slot]),
            pltpu.make_async_copy(ae_h.at[pl.ds(off, _K)], aev.at[slot],
                                  semi.at[slot]),
        )

    def gather_copy(g):
        b = lax.rem(g, 2)
        slot = lax.rem(g, 3)
        return pltpu.make_async_copy(xp_h.at[srcv.at[slot]], rows.at[b],
                                     semg.at[b])

    def scatter_copies(g):
        b = lax.rem(g, 2)
        slot = lax.rem(g, 3)
        return (
            pltpu.make_async_copy(rows.at[b], outsp.at[dstv.at[slot]],
                                  sems.at[b]),
            pltpu.make_async_copy(pv.at[b], densp.at[dstv.at[slot]],
                                  sems.at[b]),
        )

    for cp in idx_copies(0):
        cp.start()
    for cp in idx_copies(0):
        cp.wait()
    gather_copy(0).start()
    for cp in idx_copies(1):
        cp.start()

    def chunk(g, carry):
        b = lax.rem(g, 2)
        slot = lax.rem(g, 3)
        for i in range(_K // 16):
            sl = pl.ds(i * 16, 16)
            av = plsc.load_gather(asv, [srcv[slot, sl]])
            bv = plsc.load_gather(adv, [dstv[slot, sl]])
            al = _lrelu(av + bv + aev[slot, sl])
            pv[b, sl] = jnp.exp(al)
        gather_copy(g).wait()

        bvec = jnp.full((16,), b, jnp.int32)

        def scale(i, cc):
            p = plsc.load_gather(pv, [bvec, jnp.full((16,), i, jnp.int32)])
            for j in range(_D // 16):
                sl2 = pl.ds(j * 16, 16)
                rows[b, i, sl2] = rows[b, i, sl2] * p
            return cc

        lax.fori_loop(0, _K, scale, 0)

        @pl.when(g >= 1)
        def _():
            for cp in scatter_copies(g - 1):
                cp.wait()

        @pl.when(g + 1 < _CH)
        def _():
            for cp in idx_copies(g + 1):
                cp.wait()
            gather_copy(g + 1).start()

        @pl.when(g + 2 < _CH)
        def _():
            for cp in idx_copies(g + 2):
                cp.start()

        for cp in scatter_copies(g):
            cp.start(add=True)
        return carry

    lax.fori_loop(0, _CH, chunk, 0)
    for cp in scatter_copies(_CH - 1):
        cp.wait()
    plsc.subcore_barrier()
    pltpu.sync_copy(outsp.at[pl.ds(s * _RPT, _RPT)],
                    out_h.at[pl.ds(c * _NP + s * _RPT, _RPT)])
    pltpu.sync_copy(densp.at[pl.ds(s * _RPT, _RPT)],
                    den_h.at[pl.ds(c * _NP + s * _RPT, _RPT)])


@functools.cache
def _sc_edge():
    return pl.kernel(
        _sc_body,
        mesh=plsc.VectorSubcoreMesh(core_axis_name="c", subcore_axis_name="s"),
        compiler_params=pltpu.CompilerParams(needs_layout_passes=False),
        out_type=[
            jax.ShapeDtypeStruct((_NC * _NP, _D), jnp.float32),
            jax.ShapeDtypeStruct((_NC * _NP,), jnp.float32),
        ],
        scratch_types=[
            pltpu.VMEM((_NP,), jnp.float32),
            pltpu.VMEM((_NP,), jnp.float32),
            pltpu.VMEM((3, _K), jnp.int32),
            pltpu.VMEM((3, _K), jnp.int32),
            pltpu.VMEM((3, _K), jnp.float32),
            pltpu.VMEM((2, _K), jnp.float32),
            pltpu.VMEM((2, _K, _D), jnp.float32),
            pltpu.VMEM_SHARED((_NP, _D), jnp.float32),
            pltpu.VMEM_SHARED((_NP,), jnp.float32),
            pltpu.SemaphoreType.DMA((3,)),
            pltpu.SemaphoreType.DMA((2,)),
            pltpu.SemaphoreType.DMA((2,)),
        ],
    )


def _comb1_body(op_ref, d_ref, xp_ref, as_ref, ad_ref, m_ref, b_ref,
                w2_ref, ats_ref, atd_ref, xp2_ref, as2_ref, ad2_ref):
    ploop = jnp.exp(_lrelu(as_ref[...] + ad_ref[...] + m_ref[...]))
    num = op_ref[0] + op_ref[1] + ploop * xp_ref[...]
    den = d_ref[0] + d_ref[1] + ploop + 1e-16
    h = num / den + b_ref[...]
    h = jnp.where(h > 0, h, jnp.exp(jnp.minimum(h, 0.0)) - 1.0)
    xp2 = jnp.dot(h, w2_ref[...], preferred_element_type=jnp.float32)
    xp2_ref[...] = xp2
    as2_ref[...] = jnp.sum(xp2 * ats_ref[...], axis=-1, keepdims=True)
    ad2_ref[...] = jnp.sum(xp2 * atd_ref[...], axis=-1, keepdims=True)


def _tc_comb1(op, dp, xp, a_s, a_d, m, b, W2, ats2, atd2):
    return pl.pallas_call(
        _comb1_body,
        grid=(_GN,),
        in_specs=[
            pl.BlockSpec((_NC, _BN, _D), lambda i: (0, i, 0)),
            pl.BlockSpec((_NC, _BN, 1), lambda i: (0, i, 0)),
            pl.BlockSpec((_BN, _D), lambda i: (i, 0)),
            pl.BlockSpec((_BN, 1), lambda i: (i, 0)),
            pl.BlockSpec((_BN, 1), lambda i: (i, 0)),
            pl.BlockSpec((1, 1), lambda i: (0, 0)),
            pl.BlockSpec((1, _D), lambda i: (0, 0)),
            pl.BlockSpec((_D, _D), lambda i: (0, 0)),
            pl.BlockSpec((1, _D), lambda i: (0, 0)),
            pl.BlockSpec((1, _D), lambda i: (0, 0)),
        ],
        out_specs=[
            pl.BlockSpec((_BN, _D), lambda i: (i, 0)),
            pl.BlockSpec((_BN, 1), lambda i: (i, 0)),
            pl.BlockSpec((_BN, 1), lambda i: (i, 0)),
        ],
        out_shape=[
            jax.ShapeDtypeStruct((_NP, _D), jnp.float32),
            jax.ShapeDtypeStruct((_NP, 1), jnp.float32),
            jax.ShapeDtypeStruct((_NP, 1), jnp.float32),
        ],
    )(op, dp, xp, a_s, a_d, m, b, W2, ats2, atd2)


def _comb2_body(op_ref, d_ref, xp_ref, as_ref, ad_ref, m_ref, b_ref, o_ref):
    ploop = jnp.exp(_lrelu(as_ref[...] + ad_ref[...] + m_ref[...]))
    num = op_ref[0] + op_ref[1] + ploop * xp_ref[...]
    den = d_ref[0] + d_ref[1] + ploop + 1e-16
    o_ref[...] = num / den + b_ref[...]


def _tc_comb2(op, dp, xp, a_s, a_d, m, b):
    return pl.pallas_call(
        _comb2_body,
        grid=(_GN,),
        in_specs=[
            pl.BlockSpec((_NC, _BN, _D), lambda i: (0, i, 0)),
            pl.BlockSpec((_NC, _BN, 1), lambda i: (0, i, 0)),
            pl.BlockSpec((_BN, _D), lambda i: (i, 0)),
            pl.BlockSpec((_BN, 1), lambda i: (i, 0)),
            pl.BlockSpec((_BN, 1), lambda i: (i, 0)),
            pl.BlockSpec((1, 1), lambda i: (0, 0)),
            pl.BlockSpec((1, _D), lambda i: (0, 0)),
        ],
        out_specs=pl.BlockSpec((_BN, _D), lambda i: (i, 0)),
        out_shape=jax.ShapeDtypeStruct((_NP, _D), jnp.float32),
    )(op, dp, xp, a_s, a_d, m, b)


def kernel(x, edge_index, edge_weight, W1, W_edge1, att_src1, att_dst1,
           att_edge1, bias1, W2, W_edge2, att_src2, att_dst2, att_edge2,
           bias2):
    xpad = jnp.pad(x, ((0, _NP - _N), (0, 0)))
    src1d = edge_index[0]
    dst1d = edge_index[1]
    ew2d = edge_weight.reshape(_E // _D, _D)
    ats1 = att_src1.reshape(1, _D)
    atd1 = att_dst1.reshape(1, _D)
    ate1 = att_edge1.reshape(1, _D)
    ats2 = att_src2.reshape(1, _D)
    atd2 = att_dst2.reshape(1, _D)
    ate2 = att_edge2.reshape(1, _D)
    b1 = bias1.reshape(1, _D)
    b2 = bias2.reshape(1, _D)
    zrow = jnp.zeros((_NP, _D), jnp.float32)
    zd = jnp.zeros((_NP,), jnp.float32)

    xp1, as1, ad1, ae1, ae2, m1, m2 = _tc_prep(
        xpad, W1, ats1, atd1, ew2d, W_edge1.reshape(1, _D), ate1,
        W_edge2.reshape(1, _D), ate2)
    op1, dp1 = _sc_edge()(src1d, dst1d, ae1.reshape(_E), xp1,
                          as1.reshape(_NP), ad1.reshape(_NP), zrow, zd)
    xp2, as2, ad2 = _tc_comb1(op1.reshape(_NC, _NP, _D),
                              dp1.reshape(_NC, _NP, 1),
                              xp1, as1, ad1, m1, b1, W2, ats2, atd2)
    op2, dp2 = _sc_edge()(src1d, dst1d, ae2.reshape(_E), xp2,
                          as2.reshape(_NP), ad2.reshape(_NP), zrow, zd)
    out = _tc_comb2(op2.reshape(_NC, _NP, _D), dp2.reshape(_NC, _NP, 1),
                    xp2, as2, ad2, m2, b2)
    return out[:_N]


# speedup vs baseline: 1.8485x; 1.0318x over previous
import functools

import jax
import jax.numpy as jnp
from jax import lax
from jax.experimental import pallas as pl
from jax.experimental.pallas import tpu as pltpu
from jax.experimental.pallas import tpu_sc as plsc

_N = 10000
_NP = 10240
_E = 320000
_D = 128
_NC = 2
_NS = 16
_NW = _NC * _NS
_EPW = _E // _NW
_K = 80
_CH = _EPW // _K
_U = 5
_UPW = _CH
_RPT = _NP // _NS
_BN = 2048
_GN = _NP // _BN


def _lrelu(v):
    return jnp.where(v >= 0, v, 0.2 * v)


def _prep_body(x_ref, w_ref, ats_ref, atd_ref, ew_ref, we1_ref, ate1_ref,
               we2_ref, ate2_ref,
               xp_ref, as_ref, ad_ref, ae1_ref, ae2_ref, m1_ref, m2_ref):
    i = pl.program_id(0)
    xp = jnp.dot(x_ref[...], w_ref[...], preferred_element_type=jnp.float32)
    xp_ref[...] = xp
    as_ref[...] = jnp.sum(xp * ats_ref[...], axis=-1, keepdims=True)
    ad_ref[...] = jnp.sum(xp * atd_ref[...], axis=-1, keepdims=True)
    c1 = jnp.sum(we1_ref[...] * ate1_ref[...])
    c2 = jnp.sum(we2_ref[...] * ate2_ref[...])
    ew = ew_ref[...]
    ae1_ref[...] = ew * c1
    ae2_ref[...] = ew * c2
    @pl.when(i == 0)
    def _():
        mean = (jnp.sum(ew) * (1.0 / _E)).reshape(1, 1)
        m1_ref[...] = mean * c1
        m2_ref[...] = mean * c2


def _tc_prep(x, W, ats, atd, ew2d, we1, ate1, we2, ate2):
    eb = _E // _D
    return pl.pallas_call(
        _prep_body,
        grid=(_GN,),
        in_specs=[
            pl.BlockSpec((_BN, _D), lambda i: (i, 0)),
            pl.BlockSpec((_D, _D), lambda i: (0, 0)),
            pl.BlockSpec((1, _D), lambda i: (0, 0)),
            pl.BlockSpec((1, _D), lambda i: (0, 0)),
            pl.BlockSpec((eb, _D), lambda i: (0, 0)),
            pl.BlockSpec((1, _D), lambda i: (0, 0)),
            pl.BlockSpec((1, _D), lambda i: (0, 0)),
            pl.BlockSpec((1, _D), lambda i: (0, 0)),
            pl.BlockSpec((1, _D), lambda i: (0, 0)),
        ],
        out_specs=[
            pl.BlockSpec((_BN, _D), lambda i: (i, 0)),
            pl.BlockSpec((_BN, 1), lambda i: (i, 0)),
            pl.BlockSpec((_BN, 1), lambda i: (i, 0)),
            pl.BlockSpec((eb, _D), lambda i: (0, 0)),
            pl.BlockSpec((eb, _D), lambda i: (0, 0)),
            pl.BlockSpec((1, 1), lambda i: (0, 0)),
            pl.BlockSpec((1, 1), lambda i: (0, 0)),
        ],
        out_shape=[
            jax.ShapeDtypeStruct((_NP, _D), jnp.float32),
            jax.ShapeDtypeStruct((_NP, 1), jnp.float32),
            jax.ShapeDtypeStruct((_NP, 1), jnp.float32),
            jax.ShapeDtypeStruct((_E // _D, _D), jnp.float32),
            jax.ShapeDtypeStruct((_E // _D, _D), jnp.float32),
            jax.ShapeDtypeStruct((1, 1), jnp.float32),
            jax.ShapeDtypeStruct((1, 1), jnp.float32),
        ],
    )(x, W, ats, atd, ew2d, we1, ate1, we2, ate2)


def _sc_body(src_h, dst_h, ae_h, xp_h, as_h, ad_h, z_h, zd_h,
             out_h, den_h,
             asv, adv, srcv, dstv, aev, pv, rows, outsp, densp,
             semi, semg, sems):
    c = lax.axis_index("c")
    s = lax.axis_index("s")
    pltpu.sync_copy(z_h.at[pl.ds(s * _RPT, _RPT)], outsp.at[pl.ds(s * _RPT, _RPT)])
    pltpu.sync_copy(zd_h.at[pl.ds(s * _RPT, _RPT)], densp.at[pl.ds(s * _RPT, _RPT)])
    pltpu.sync_copy(as_h, asv)
    pltpu.sync_copy(ad_h, adv)
    plsc.subcore_barrier()

    base_e = (c * _NS + s) * _EPW

    def idx_copies(g):
        slot = lax.rem(g, 3)
        off = pl.multiple_of(base_e + g * _K, 8)
        return (
            pltpu.make_async_copy(src_h.at[pl.ds(off, _K)], srcv.at[slot],
                                  semi.at[slot]),
            pltpu.make_async_copy(dst_h.at[pl.ds(off, _K)], dstv.at[slot],
                                  semi.at[slot]),
            pltpu.make_async_copy(ae_h.at[pl.ds(off, _K)], aev.at[slot],
                                  semi.at[slot]),
        )

    def gather_copy(g):
        b = lax.rem(g, 2)
        slot = lax.rem(g, 3)
        return pltpu.make_async_copy(xp_h.at[srcv.at[slot]], rows.at[b],
                                     semg.at[b])

    def scatter_copies(g):
        b = lax.rem(g, 2)
        slot = lax.rem(g, 3)
        return (
            pltpu.make_async_copy(rows.at[b], outsp.at[dstv.at[slot]],
                                  sems.at[b]),
            pltpu.make_async_copy(pv.at[b], densp.at[dstv.at[slot]],
                                  sems.at[b]),
        )

    for cp in idx_copies(0):
        cp.start()
    for cp in idx_copies(0):
        cp.wait()
    gather_copy(0).start()
    for cp in idx_copies(1):
        cp.start()

    def chunk(g, carry):
        b = lax.rem(g, 2)
        slot = lax.rem(g, 3)
        for i in range(_K // 16):
            sl = pl.ds(i * 16, 16)
            av = plsc.load_gather(asv, [srcv[slot, sl]])
            bv = plsc.load_gather(adv, [dstv[slot, sl]])
            al = _lrelu(av + bv + aev[slot, sl])
            pv[b, sl] = jnp.exp(al)
        gather_copy(g).wait()

        bvec = jnp.full((16,), b, jnp.int32)

        def scale(i, cc):
            p = plsc.load_gather(pv, [bvec, jnp.full((16,), i, jnp.int32)])
            for j in range(_D // 16):
                sl2 = pl.ds(j * 16, 16)
                rows[b, i, sl2] = rows[b, i, sl2] * p
            return cc

        lax.fori_loop(0, _K, scale, 0, unroll=8)

        @pl.when(g >= 1)
        def _():
            for cp in scatter_copies(g - 1):
                cp.wait()

        @pl.when(g + 1 < _CH)
        def _():
            for cp in idx_copies(g + 1):
                cp.wait()
            gather_copy(g + 1).start()

        @pl.when(g + 2 < _CH)
        def _():
            for cp in idx_copies(g + 2):
                cp.start()

        for cp in scatter_copies(g):
            cp.start(add=True)
        return carry

    lax.fori_loop(0, _CH, chunk, 0)
    for cp in scatter_copies(_CH - 1):
        cp.wait()
    plsc.subcore_barrier()
    pltpu.sync_copy(outsp.at[pl.ds(s * _RPT, _RPT)],
                    out_h.at[pl.ds(c * _NP + s * _RPT, _RPT)])
    pltpu.sync_copy(densp.at[pl.ds(s * _RPT, _RPT)],
                    den_h.at[pl.ds(c * _NP + s * _RPT, _RPT)])


@functools.cache
def _sc_edge():
    return pl.kernel(
        _sc_body,
        mesh=plsc.VectorSubcoreMesh(core_axis_name="c", subcore_axis_name="s"),
        compiler_params=pltpu.CompilerParams(needs_layout_passes=False),
        out_type=[
            jax.ShapeDtypeStruct((_NC * _NP, _D), jnp.float32),
            jax.ShapeDtypeStruct((_NC * _NP,), jnp.float32),
        ],
        scratch_types=[
            pltpu.VMEM((_NP,), jnp.float32),
            pltpu.VMEM((_NP,), jnp.float32),
            pltpu.VMEM((3, _K), jnp.int32),
            pltpu.VMEM((3, _K), jnp.int32),
            pltpu.VMEM((3, _K), jnp.float32),
            pltpu.VMEM((2, _K), jnp.float32),
            pltpu.VMEM((2, _K, _D), jnp.float32),
            pltpu.VMEM_SHARED((_NP, _D), jnp.float32),
            pltpu.VMEM_SHARED((_NP,), jnp.float32),
            pltpu.SemaphoreType.DMA((3,)),
            pltpu.SemaphoreType.DMA((2,)),
            pltpu.SemaphoreType.DMA((2,)),
        ],
    )


def _comb1_body(op_ref, d_ref, xp_ref, as_ref, ad_ref, m_ref, b_ref,
                w2_ref, ats_ref, atd_ref, xp2_ref, as2_ref, ad2_ref):
    ploop = jnp.exp(_lrelu(as_ref[...] + ad_ref[...] + m_ref[...]))
    num = op_ref[0] + op_ref[1] + ploop * xp_ref[...]
    den = d_ref[0] + d_ref[1] + ploop + 1e-16
    h = num / den + b_ref[...]
    h = jnp.where(h > 0, h, jnp.exp(jnp.minimum(h, 0.0)) - 1.0)
    xp2 = jnp.dot(h, w2_ref[...], preferred_element_type=jnp.float32)
    xp2_ref[...] = xp2
    as2_ref[...] = jnp.sum(xp2 * ats_ref[...], axis=-1, keepdims=True)
    ad2_ref[...] = jnp.sum(xp2 * atd_ref[...], axis=-1, keepdims=True)


def _tc_comb1(op, dp, xp, a_s, a_d, m, b, W2, ats2, atd2):
    return pl.pallas_call(
        _comb1_body,
        grid=(_GN,),
        in_specs=[
            pl.BlockSpec((_NC, _BN, _D), lambda i: (0, i, 0)),
            pl.BlockSpec((_NC, _BN, 1), lambda i: (0, i, 0)),
            pl.BlockSpec((_BN, _D), lambda i: (i, 0)),
            pl.BlockSpec((_BN, 1), lambda i: (i, 0)),
            pl.BlockSpec((_BN, 1), lambda i: (i, 0)),
            pl.BlockSpec((1, 1), lambda i: (0, 0)),
            pl.BlockSpec((1, _D), lambda i: (0, 0)),
            pl.BlockSpec((_D, _D), lambda i: (0, 0)),
            pl.BlockSpec((1, _D), lambda i: (0, 0)),
            pl.BlockSpec((1, _D), lambda i: (0, 0)),
        ],
        out_specs=[
            pl.BlockSpec((_BN, _D), lambda i: (i, 0)),
            pl.BlockSpec((_BN, 1), lambda i: (i, 0)),
            pl.BlockSpec((_BN, 1), lambda i: (i, 0)),
        ],
        out_shape=[
            jax.ShapeDtypeStruct((_NP, _D), jnp.float32),
            jax.ShapeDtypeStruct((_NP, 1), jnp.float32),
            jax.ShapeDtypeStruct((_NP, 1), jnp.float32),
        ],
    )(op, dp, xp, a_s, a_d, m, b, W2, ats2, atd2)


def _comb2_body(op_ref, d_ref, xp_ref, as_ref, ad_ref, m_ref, b_ref, o_ref):
    ploop = jnp.exp(_lrelu(as_ref[...] + ad_ref[...] + m_ref[...]))
    num = op_ref[0] + op_ref[1] + ploop * xp_ref[...]
    den = d_ref[0] + d_ref[1] + ploop + 1e-16
    o_ref[...] = num / den + b_ref[...]


def _tc_comb2(op, dp, xp, a_s, a_d, m, b):
    return pl.pallas_call(
        _comb2_body,
        grid=(_GN,),
        in_specs=[
            pl.BlockSpec((_NC, _BN, _D), lambda i: (0, i, 0)),
            pl.BlockSpec((_NC, _BN, 1), lambda i: (0, i, 0)),
            pl.BlockSpec((_BN, _D), lambda i: (i, 0)),
            pl.BlockSpec((_BN, 1), lambda i: (i, 0)),
            pl.BlockSpec((_BN, 1), lambda i: (i, 0)),
            pl.BlockSpec((1, 1), lambda i: (0, 0)),
            pl.BlockSpec((1, _D), lambda i: (0, 0)),
        ],
        out_specs=pl.BlockSpec((_BN, _D), lambda i: (i, 0)),
        out_shape=jax.ShapeDtypeStruct((_NP, _D), jnp.float32),
    )(op, dp, xp, a_s, a_d, m, b)


def kernel(x, edge_index, edge_weight, W1, W_edge1, att_src1, att_dst1,
           att_edge1, bias1, W2, W_edge2, att_src2, att_dst2, att_edge2,
           bias2):
    xpad = jnp.pad(x, ((0, _NP - _N), (0, 0)))
    src1d = edge_index[0]
    dst1d = edge_index[1]
    ew2d = edge_weight.reshape(_E // _D, _D)
    ats1 = att_src1.reshape(1, _D)
    atd1 = att_dst1.reshape(1, _D)
    ate1 = att_edge1.reshape(1, _D)
    ats2 = att_src2.reshape(1, _D)
    atd2 = att_dst2.reshape(1, _D)
    ate2 = att_edge2.reshape(1, _D)
    b1 = bias1.reshape(1, _D)
    b2 = bias2.reshape(1, _D)
    zrow = jnp.zeros((_NP, _D), jnp.float32)
    zd = jnp.zeros((_NP,), jnp.float32)

    xp1, as1, ad1, ae1, ae2, m1, m2 = _tc_prep(
        xpad, W1, ats1, atd1, ew2d, W_edge1.reshape(1, _D), ate1,
        W_edge2.reshape(1, _D), ate2)
    op1, dp1 = _sc_edge()(src1d, dst1d, ae1.reshape(_E), xp1,
                          as1.reshape(_NP), ad1.reshape(_NP), zrow, zd)
    xp2, as2, ad2 = _tc_comb1(op1.reshape(_NC, _NP, _D),
                              dp1.reshape(_NC, _NP, 1),
                              xp1, as1, ad1, m1, b1, W2, ats2, atd2)
    op2, dp2 = _sc_edge()(src1d, dst1d, ae2.reshape(_E), xp2,
                          as2.reshape(_NP), ad2.reshape(_NP), zrow, zd)
    out = _tc_comb2(op2.reshape(_NC, _NP, _D), dp2.reshape(_NC, _NP, 1),
                    xp2, as2, ad2, m2, b2)
    return out[:_N]


# speedup vs baseline: 2.4252x; 1.3120x over previous
import functools

import jax
import jax.numpy as jnp
from jax import lax
from jax.experimental import pallas as pl
from jax.experimental.pallas import tpu as pltpu
from jax.experimental.pallas import tpu_sc as plsc

_N = 10000
_NP = 10240
_E = 320000
_D = 128
_NC = 2
_NS = 16
_NW = _NC * _NS
_EPW = _E // _NW
_K = 80
_CH = _EPW // _K
_U = 5
_UPW = _CH
_RPT = _NP // _NS
_BN = 2048
_GN = _NP // _BN


def _lrelu(v):
    return jnp.where(v >= 0, v, 0.2 * v)


def _prep_body(x_ref, w_ref, ats_ref, atd_ref, ew_ref, we1_ref, ate1_ref,
               we2_ref, ate2_ref,
               xp_ref, as_ref, ad_ref, ae1_ref, ae2_ref, m1_ref, m2_ref):
    i = pl.program_id(0)
    xp = jnp.dot(x_ref[...], w_ref[...], preferred_element_type=jnp.float32)
    xp_ref[...] = xp
    as_ref[...] = jnp.sum(xp * ats_ref[...], axis=-1, keepdims=True)
    ad_ref[...] = jnp.sum(xp * atd_ref[...], axis=-1, keepdims=True)
    c1 = jnp.sum(we1_ref[...] * ate1_ref[...])
    c2 = jnp.sum(we2_ref[...] * ate2_ref[...])
    ew = ew_ref[...]
    ae1_ref[...] = ew * c1
    ae2_ref[...] = ew * c2
    @pl.when(i == 0)
    def _():
        mean = (jnp.sum(ew) * (1.0 / _E)).reshape(1, 1)
        m1_ref[...] = mean * c1
        m2_ref[...] = mean * c2


def _tc_prep(x, W, ats, atd, ew2d, we1, ate1, we2, ate2):
    eb = _E // _D
    return pl.pallas_call(
        _prep_body,
        grid=(_GN,),
        in_specs=[
            pl.BlockSpec((_BN, _D), lambda i: (i, 0)),
            pl.BlockSpec((_D, _D), lambda i: (0, 0)),
            pl.BlockSpec((1, _D), lambda i: (0, 0)),
            pl.BlockSpec((1, _D), lambda i: (0, 0)),
            pl.BlockSpec((eb, _D), lambda i: (0, 0)),
            pl.BlockSpec((1, _D), lambda i: (0, 0)),
            pl.BlockSpec((1, _D), lambda i: (0, 0)),
            pl.BlockSpec((1, _D), lambda i: (0, 0)),
            pl.BlockSpec((1, _D), lambda i: (0, 0)),
        ],
        out_specs=[
            pl.BlockSpec((_BN, _D), lambda i: (i, 0)),
            pl.BlockSpec((_BN, 1), lambda i: (i, 0)),
            pl.BlockSpec((_BN, 1), lambda i: (i, 0)),
            pl.BlockSpec((eb, _D), lambda i: (0, 0)),
            pl.BlockSpec((eb, _D), lambda i: (0, 0)),
            pl.BlockSpec((1, 1), lambda i: (0, 0)),
            pl.BlockSpec((1, 1), lambda i: (0, 0)),
        ],
        out_shape=[
            jax.ShapeDtypeStruct((_NP, _D), jnp.float32),
            jax.ShapeDtypeStruct((_NP, 1), jnp.float32),
            jax.ShapeDtypeStruct((_NP, 1), jnp.float32),
            jax.ShapeDtypeStruct((_E // _D, _D), jnp.float32),
            jax.ShapeDtypeStruct((_E // _D, _D), jnp.float32),
            jax.ShapeDtypeStruct((1, 1), jnp.float32),
            jax.ShapeDtypeStruct((1, 1), jnp.float32),
        ],
    )(x, W, ats, atd, ew2d, we1, ate1, we2, ate2)


def _sc_body(src_h, dst_h, ae_h, xp_h, as_h, ad_h, z_h, zd_h,
             out_h, den_h,
             asv, adv, srcv, dstv, aev, pv, rows, outsp, densp,
             semi, semg, sems):
    c = lax.axis_index("c")
    s = lax.axis_index("s")
    pltpu.sync_copy(z_h.at[pl.ds(s * _RPT, _RPT)], outsp.at[pl.ds(s * _RPT, _RPT)])
    pltpu.sync_copy(zd_h.at[pl.ds(s * _RPT, _RPT)], densp.at[pl.ds(s * _RPT, _RPT)])
    pltpu.sync_copy(as_h, asv)
    pltpu.sync_copy(ad_h, adv)
    plsc.subcore_barrier()

    base_e = (c * _NS + s) * _EPW

    def idx_copies(g):
        slot = lax.rem(g, 3)
        off = pl.multiple_of(base_e + g * _K, 8)
        return (
            pltpu.make_async_copy(src_h.at[pl.ds(off, _K)], srcv.at[slot],
                                  semi.at[slot]),
            pltpu.make_async_copy(dst_h.at[pl.ds(off, _K)], dstv.at[slot],
                                  semi.at[slot]),
            pltpu.make_async_copy(ae_h.at[pl.ds(off, _K)], aev.at[slot],
                                  semi.at[slot]),
        )

    def gather_copy(g):
        b = lax.rem(g, 2)
        slot = lax.rem(g, 3)
        return pltpu.make_async_copy(xp_h.at[srcv.at[slot]], rows.at[b],
                                     semg.at[b])

    def scatter_copies(g):
        b = lax.rem(g, 2)
        slot = lax.rem(g, 3)
        return (
            pltpu.make_async_copy(rows.at[b], outsp.at[dstv.at[slot]],
                                  sems.at[b]),
            pltpu.make_async_copy(pv.at[b], densp.at[dstv.at[slot]],
                                  sems.at[b]),
        )

    for cp in idx_copies(0):
        cp.start()
    for cp in idx_copies(0):
        cp.wait()
    gather_copy(0).start()
    for cp in idx_copies(1):
        cp.start()

    def chunk(g, carry):
        b = lax.rem(g, 2)
        slot = lax.rem(g, 3)
        for i in range(_K // 16):
            sl = pl.ds(i * 16, 16)
            av = plsc.load_gather(asv, [srcv[slot, sl]])
            bv = plsc.load_gather(adv, [dstv[slot, sl]])
            al = _lrelu(av + bv + aev[slot, sl])
            pv[b, sl] = jnp.exp(al)
        gather_copy(g).wait()

        @pl.when(g >= 1)
        def _():
            for cp in scatter_copies(g - 1):
                cp.wait()

        @pl.when(g + 1 < _CH)
        def _():
            for cp in idx_copies(g + 1):
                cp.wait()
            gather_copy(g + 1).start()

        @pl.when(g + 2 < _CH)
        def _():
            for cp in idx_copies(g + 2):
                cp.start()

        bvec = jnp.full((16,), b, jnp.int32)

        def scale(i, cc):
            p = plsc.load_gather(pv, [bvec, jnp.full((16,), i, jnp.int32)])
            for j in range(_D // 16):
                sl2 = pl.ds(j * 16, 16)
                rows[b, i, sl2] = rows[b, i, sl2] * p
            return cc

        lax.fori_loop(0, _K, scale, 0, unroll=8)

        for cp in scatter_copies(g):
            cp.start(add=True)
        return carry

    lax.fori_loop(0, _CH, chunk, 0)
    for cp in scatter_copies(_CH - 1):
        cp.wait()
    plsc.subcore_barrier()
    pltpu.sync_copy(outsp.at[pl.ds(s * _RPT, _RPT)],
                    out_h.at[pl.ds(c * _NP + s * _RPT, _RPT)])
    pltpu.sync_copy(densp.at[pl.ds(s * _RPT, _RPT)],
                    den_h.at[pl.ds(c * _NP + s * _RPT, _RPT)])


@functools.cache
def _sc_edge():
    return pl.kernel(
        _sc_body,
        mesh=plsc.VectorSubcoreMesh(core_axis_name="c", subcore_axis_name="s"),
        compiler_params=pltpu.CompilerParams(needs_layout_passes=False),
        out_type=[
            jax.ShapeDtypeStruct((_NC * _NP, _D), jnp.float32),
            jax.ShapeDtypeStruct((_NC * _NP,), jnp.float32),
        ],
        scratch_types=[
            pltpu.VMEM((_NP,), jnp.float32),
            pltpu.VMEM((_NP,), jnp.float32),
            pltpu.VMEM((3, _K), jnp.int32),
            pltpu.VMEM((3, _K), jnp.int32),
            pltpu.VMEM((3, _K), jnp.float32),
            pltpu.VMEM((2, _K), jnp.float32),
            pltpu.VMEM((2, _K, _D), jnp.float32),
            pltpu.VMEM_SHARED((_NP, _D), jnp.float32),
            pltpu.VMEM_SHARED((_NP,), jnp.float32),
            pltpu.SemaphoreType.DMA((3,)),
            pltpu.SemaphoreType.DMA((2,)),
            pltpu.SemaphoreType.DMA((2,)),
        ],
    )


def _comb1_body(op_ref, d_ref, xp_ref, as_ref, ad_ref, m_ref, b_ref,
                w2_ref, ats_ref, atd_ref, xp2_ref, as2_ref, ad2_ref):
    ploop = jnp.exp(_lrelu(as_ref[...] + ad_ref[...] + m_ref[...]))
    num = op_ref[0] + op_ref[1] + ploop * xp_ref[...]
    den = d_ref[0] + d_ref[1] + ploop + 1e-16
    h = num / den + b_ref[...]
    h = jnp.where(h > 0, h, jnp.exp(jnp.minimum(h, 0.0)) - 1.0)
    xp2 = jnp.dot(h, w2_ref[...], preferred_element_type=jnp.float32)
    xp2_ref[...] = xp2
    as2_ref[...] = jnp.sum(xp2 * ats_ref[...], axis=-1, keepdims=True)
    ad2_ref[...] = jnp.sum(xp2 * atd_ref[...], axis=-1, keepdims=True)


def _tc_comb1(op, dp, xp, a_s, a_d, m, b, W2, ats2, atd2):
    return pl.pallas_call(
        _comb1_body,
        grid=(_GN,),
        in_specs=[
            pl.BlockSpec((_NC, _BN, _D), lambda i: (0, i, 0)),
            pl.BlockSpec((_NC, _BN, 1), lambda i: (0, i, 0)),
            pl.BlockSpec((_BN, _D), lambda i: (i, 0)),
            pl.BlockSpec((_BN, 1), lambda i: (i, 0)),
            pl.BlockSpec((_BN, 1), lambda i: (i, 0)),
            pl.BlockSpec((1, 1), lambda i: (0, 0)),
            pl.BlockSpec((1, _D), lambda i: (0, 0)),
            pl.BlockSpec((_D, _D), lambda i: (0, 0)),
            pl.BlockSpec((1, _D), lambda i: (0, 0)),
            pl.BlockSpec((1, _D), lambda i: (0, 0)),
        ],
        out_specs=[
            pl.BlockSpec((_BN, _D), lambda i: (i, 0)),
            pl.BlockSpec((_BN, 1), lambda i: (i, 0)),
            pl.BlockSpec((_BN, 1), lambda i: (i, 0)),
        ],
        out_shape=[
            jax.ShapeDtypeStruct((_NP, _D), jnp.float32),
            jax.ShapeDtypeStruct((_NP, 1), jnp.float32),
            jax.ShapeDtypeStruct((_NP, 1), jnp.float32),
        ],
    )(op, dp, xp, a_s, a_d, m, b, W2, ats2, atd2)


def _comb2_body(op_ref, d_ref, xp_ref, as_ref, ad_ref, m_ref, b_ref, o_ref):
    ploop = jnp.exp(_lrelu(as_ref[...] + ad_ref[...] + m_ref[...]))
    num = op_ref[0] + op_ref[1] + ploop * xp_ref[...]
    den = d_ref[0] + d_ref[1] + ploop + 1e-16
    o_ref[...] = num / den + b_ref[...]


def _tc_comb2(op, dp, xp, a_s, a_d, m, b):
    return pl.pallas_call(
        _comb2_body,
        grid=(_GN,),
        in_specs=[
            pl.BlockSpec((_NC, _BN, _D), lambda i: (0, i, 0)),
            pl.BlockSpec((_NC, _BN, 1), lambda i: (0, i, 0)),
            pl.BlockSpec((_BN, _D), lambda i: (i, 0)),
            pl.BlockSpec((_BN, 1), lambda i: (i, 0)),
            pl.BlockSpec((_BN, 1), lambda i: (i, 0)),
            pl.BlockSpec((1, 1), lambda i: (0, 0)),
            pl.BlockSpec((1, _D), lambda i: (0, 0)),
        ],
        out_specs=pl.BlockSpec((_BN, _D), lambda i: (i, 0)),
        out_shape=jax.ShapeDtypeStruct((_NP, _D), jnp.float32),
    )(op, dp, xp, a_s, a_d, m, b)


def kernel(x, edge_index, edge_weight, W1, W_edge1, att_src1, att_dst1,
           att_edge1, bias1, W2, W_edge2, att_src2, att_dst2, att_edge2,
           bias2):
    xpad = jnp.pad(x, ((0, _NP - _N), (0, 0)))
    src1d = edge_index[0]
    dst1d = edge_index[1]
    ew2d = edge_weight.reshape(_E // _D, _D)
    ats1 = att_src1.reshape(1, _D)
    atd1 = att_dst1.reshape(1, _D)
    ate1 = att_edge1.reshape(1, _D)
    ats2 = att_src2.reshape(1, _D)
    atd2 = att_dst2.reshape(1, _D)
    ate2 = att_edge2.reshape(1, _D)
    b1 = bias1.reshape(1, _D)
    b2 = bias2.reshape(1, _D)
    zrow = jnp.zeros((_NP, _D), jnp.float32)
    zd = jnp.zeros((_NP,), jnp.float32)

    xp1, as1, ad1, ae1, ae2, m1, m2 = _tc_prep(
        xpad, W1, ats1, atd1, ew2d, W_edge1.reshape(1, _D), ate1,
        W_edge2.reshape(1, _D), ate2)
    op1, dp1 = _sc_edge()(src1d, dst1d, ae1.reshape(_E), xp1,
                          as1.reshape(_NP), ad1.reshape(_NP), zrow, zd)
    xp2, as2, ad2 = _tc_comb1(op1.reshape(_NC, _NP, _D),
                              dp1.reshape(_NC, _NP, 1),
                              xp1, as1, ad1, m1, b1, W2, ats2, atd2)
    op2, dp2 = _sc_edge()(src1d, dst1d, ae2.reshape(_E), xp2,
                          as2.reshape(_NP), ad2.reshape(_NP), zrow, zd)
    out = _tc_comb2(op2.reshape(_NC, _NP, _D), dp2.reshape(_NC, _NP, 1),
                    xp2, as2, ad2, m2, b2)
    return out[:_N]


# speedup vs baseline: 2.4341x; 1.0037x over previous
import functools

import jax
import jax.numpy as jnp
from jax import lax
from jax.experimental import pallas as pl
from jax.experimental.pallas import tpu as pltpu
from jax.experimental.pallas import tpu_sc as plsc

_N = 10000
_NP = 10240
_E = 320000
_D = 128
_NC = 2
_NS = 16
_NW = _NC * _NS
_EPW = _E // _NW
_K = 80
_CH = _EPW // _K
_U = 5
_UPW = _CH
_RPT = _NP // _NS
_BN = 2048
_GN = _NP // _BN


def _lrelu(v):
    return jnp.where(v >= 0, v, 0.2 * v)


def _prep_body(x_ref, w_ref, ats_ref, atd_ref, ew_ref, we1_ref, ate1_ref,
               we2_ref, ate2_ref,
               xp_ref, as_ref, ad_ref, ae1_ref, ae2_ref, m1_ref, m2_ref):
    i = pl.program_id(0)
    xp = jnp.dot(x_ref[...], w_ref[...], preferred_element_type=jnp.float32)
    xp_ref[...] = xp
    as_ref[...] = jnp.sum(xp * ats_ref[...], axis=-1, keepdims=True)
    ad_ref[...] = jnp.sum(xp * atd_ref[...], axis=-1, keepdims=True)
    c1 = jnp.sum(we1_ref[...] * ate1_ref[...])
    c2 = jnp.sum(we2_ref[...] * ate2_ref[...])
    ew = ew_ref[...]
    ae1_ref[...] = ew * c1
    ae2_ref[...] = ew * c2
    @pl.when(i == 0)
    def _():
        mean = (jnp.sum(ew) * (1.0 / _E)).reshape(1, 1)
        m1_ref[...] = mean * c1
        m2_ref[...] = mean * c2


def _tc_prep(x, W, ats, atd, ew2d, we1, ate1, we2, ate2):
    eb = _E // _D
    return pl.pallas_call(
        _prep_body,
        grid=(_GN,),
        in_specs=[
            pl.BlockSpec((_BN, _D), lambda i: (i, 0)),
            pl.BlockSpec((_D, _D), lambda i: (0, 0)),
            pl.BlockSpec((1, _D), lambda i: (0, 0)),
            pl.BlockSpec((1, _D), lambda i: (0, 0)),
            pl.BlockSpec((eb, _D), lambda i: (0, 0)),
            pl.BlockSpec((1, _D), lambda i: (0, 0)),
            pl.BlockSpec((1, _D), lambda i: (0, 0)),
            pl.BlockSpec((1, _D), lambda i: (0, 0)),
            pl.BlockSpec((1, _D), lambda i: (0, 0)),
        ],
        out_specs=[
            pl.BlockSpec((_BN, _D), lambda i: (i, 0)),
            pl.BlockSpec((_BN, 1), lambda i: (i, 0)),
            pl.BlockSpec((_BN, 1), lambda i: (i, 0)),
            pl.BlockSpec((eb, _D), lambda i: (0, 0)),
            pl.BlockSpec((eb, _D), lambda i: (0, 0)),
            pl.BlockSpec((1, 1), lambda i: (0, 0)),
            pl.BlockSpec((1, 1), lambda i: (0, 0)),
        ],
        out_shape=[
            jax.ShapeDtypeStruct((_NP, _D), jnp.float32),
            jax.ShapeDtypeStruct((_NP, 1), jnp.float32),
            jax.ShapeDtypeStruct((_NP, 1), jnp.float32),
            jax.ShapeDtypeStruct((_E // _D, _D), jnp.float32),
            jax.ShapeDtypeStruct((_E // _D, _D), jnp.float32),
            jax.ShapeDtypeStruct((1, 1), jnp.float32),
            jax.ShapeDtypeStruct((1, 1), jnp.float32),
        ],
    )(x, W, ats, atd, ew2d, we1, ate1, we2, ate2)


def _sc_body(src_h, dst_h, ae_h, xp_h, as_h, ad_h, z_h, zd_h,
             out_h, den_h,
             asv, adv, srcv, dstv, aev, pv, rows, outsp, densp,
             semi, semg, sems):
    c = lax.axis_index("c")
    s = lax.axis_index("s")
    pltpu.sync_copy(z_h.at[pl.ds(s * _RPT, _RPT)], outsp.at[pl.ds(s * _RPT, _RPT)])
    pltpu.sync_copy(zd_h.at[pl.ds(s * _RPT, _RPT)], densp.at[pl.ds(s * _RPT, _RPT)])
    pltpu.sync_copy(as_h, asv)
    pltpu.sync_copy(ad_h, adv)
    plsc.subcore_barrier()

    base_e = (c * _NS + s) * _EPW

    def idx_copies(g):
        slot = lax.rem(g, 3)
        off = pl.multiple_of(base_e + g * _K, 8)
        return (
            pltpu.make_async_copy(src_h.at[pl.ds(off, _K)], srcv.at[slot],
                                  semi.at[slot]),
            pltpu.make_async_copy(dst_h.at[pl.ds(off, _K)], dstv.at[slot],
                                  semi.at[slot]),
            pltpu.make_async_copy(ae_h.at[pl.ds(off, _K)], aev.at[slot],
                                  semi.at[slot]),
        )

    def gather_copy(g):
        b = lax.rem(g, 2)
        slot = lax.rem(g, 3)
        return pltpu.make_async_copy(xp_h.at[srcv.at[slot]], rows.at[b],
                                     semg.at[b])

    def scatter_copies(g):
        b = lax.rem(g, 2)
        slot = lax.rem(g, 3)
        return (
            pltpu.make_async_copy(rows.at[b], outsp.at[dstv.at[slot]],
                                  sems.at[b]),
            pltpu.make_async_copy(pv.at[b], densp.at[dstv.at[slot]],
                                  sems.at[b]),
        )

    for cp in idx_copies(0):
        cp.start()
    for cp in idx_copies(0):
        cp.wait()
    gather_copy(0).start()
    for cp in idx_copies(1):
        cp.start()

    def chunk(g, carry):
        b = lax.rem(g, 2)
        slot = lax.rem(g, 3)
        for i in range(_K // 16):
            sl = pl.ds(i * 16, 16)
            av = plsc.load_gather(asv, [srcv[slot, sl]])
            bv = plsc.load_gather(adv, [dstv[slot, sl]])
            al = _lrelu(av + bv + aev[slot, sl])
            pv[b, sl] = jnp.exp(al)
        gather_copy(g).wait()

        @pl.when(g >= 1)
        def _():
            for cp in scatter_copies(g - 1):
                cp.wait()

        @pl.when(g + 1 < _CH)
        def _():
            for cp in idx_copies(g + 1):
                cp.wait()
            gather_copy(g + 1).start()

        @pl.when(g + 2 < _CH)
        def _():
            for cp in idx_copies(g + 2):
                cp.start()

        bvec = jnp.full((16,), b, jnp.int32)

        def scale(i, cc):
            p = plsc.load_gather(pv, [bvec, jnp.full((16,), i, jnp.int32)])
            for j in range(_D // 16):
                sl2 = pl.ds(j * 16, 16)
                rows[b, i, sl2] = rows[b, i, sl2] * p
            return cc

        lax.fori_loop(0, _K, scale, 0, unroll=8)

        for cp in scatter_copies(g):
            cp.start(add=True)
        return carry

    lax.fori_loop(0, _CH, chunk, 0, unroll=2)
    for cp in scatter_copies(_CH - 1):
        cp.wait()
    plsc.subcore_barrier()
    pltpu.sync_copy(outsp.at[pl.ds(s * _RPT, _RPT)],
                    out_h.at[pl.ds(c * _NP + s * _RPT, _RPT)])
    pltpu.sync_copy(densp.at[pl.ds(s * _RPT, _RPT)],
                    den_h.at[pl.ds(c * _NP + s * _RPT, _RPT)])


@functools.cache
def _sc_edge():
    return pl.kernel(
        _sc_body,
        mesh=plsc.VectorSubcoreMesh(core_axis_name="c", subcore_axis_name="s"),
        compiler_params=pltpu.CompilerParams(needs_layout_passes=False),
        out_type=[
            jax.ShapeDtypeStruct((_NC * _NP, _D), jnp.float32),
            jax.ShapeDtypeStruct((_NC * _NP,), jnp.float32),
        ],
        scratch_types=[
            pltpu.VMEM((_NP,), jnp.float32),
            pltpu.VMEM((_NP,), jnp.float32),
            pltpu.VMEM((3, _K), jnp.int32),
            pltpu.VMEM((3, _K), jnp.int32),
            pltpu.VMEM((3, _K), jnp.float32),
            pltpu.VMEM((2, _K), jnp.float32),
            pltpu.VMEM((2, _K, _D), jnp.float32),
            pltpu.VMEM_SHARED((_NP, _D), jnp.float32),
            pltpu.VMEM_SHARED((_NP,), jnp.float32),
            pltpu.SemaphoreType.DMA((3,)),
            pltpu.SemaphoreType.DMA((2,)),
            pltpu.SemaphoreType.DMA((2,)),
        ],
    )


def _comb1_body(op_ref, d_ref, xp_ref, as_ref, ad_ref, m_ref, b_ref,
                w2_ref, ats_ref, atd_ref, xp2_ref, as2_ref, ad2_ref):
    ploop = jnp.exp(_lrelu(as_ref[...] + ad_ref[...] + m_ref[...]))
    num = op_ref[0] + op_ref[1] + ploop * xp_ref[...]
    den = d_ref[0] + d_ref[1] + ploop + 1e-16
    h = num / den + b_ref[...]
    h = jnp.where(h > 0, h, jnp.exp(jnp.minimum(h, 0.0)) - 1.0)
    xp2 = jnp.dot(h, w2_ref[...], preferred_element_type=jnp.float32)
    xp2_ref[...] = xp2
    as2_ref[...] = jnp.sum(xp2 * ats_ref[...], axis=-1, keepdims=True)
    ad2_ref[...] = jnp.sum(xp2 * atd_ref[...], axis=-1, keepdims=True)


def _tc_comb1(op, dp, xp, a_s, a_d, m, b, W2, ats2, atd2):
    return pl.pallas_call(
        _comb1_body,
        grid=(_GN,),
        in_specs=[
            pl.BlockSpec((_NC, _BN, _D), lambda i: (0, i, 0)),
            pl.BlockSpec((_NC, _BN, 1), lambda i: (0, i, 0)),
            pl.BlockSpec((_BN, _D), lambda i: (i, 0)),
            pl.BlockSpec((_BN, 1), lambda i: (i, 0)),
            pl.BlockSpec((_BN, 1), lambda i: (i, 0)),
            pl.BlockSpec((1, 1), lambda i: (0, 0)),
            pl.BlockSpec((1, _D), lambda i: (0, 0)),
            pl.BlockSpec((_D, _D), lambda i: (0, 0)),
            pl.BlockSpec((1, _D), lambda i: (0, 0)),
            pl.BlockSpec((1, _D), lambda i: (0, 0)),
        ],
        out_specs=[
            pl.BlockSpec((_BN, _D), lambda i: (i, 0)),
            pl.BlockSpec((_BN, 1), lambda i: (i, 0)),
            pl.BlockSpec((_BN, 1), lambda i: (i, 0)),
        ],
        out_shape=[
            jax.ShapeDtypeStruct((_NP, _D), jnp.float32),
            jax.ShapeDtypeStruct((_NP, 1), jnp.float32),
            jax.ShapeDtypeStruct((_NP, 1), jnp.float32),
        ],
    )(op, dp, xp, a_s, a_d, m, b, W2, ats2, atd2)


def _comb2_body(op_ref, d_ref, xp_ref, as_ref, ad_ref, m_ref, b_ref, o_ref):
    ploop = jnp.exp(_lrelu(as_ref[...] + ad_ref[...] + m_ref[...]))
    num = op_ref[0] + op_ref[1] + ploop * xp_ref[...]
    den = d_ref[0] + d_ref[1] + ploop + 1e-16
    o_ref[...] = num / den + b_ref[...]


def _tc_comb2(op, dp, xp, a_s, a_d, m, b):
    return pl.pallas_call(
        _comb2_body,
        grid=(_GN,),
        in_specs=[
            pl.BlockSpec((_NC, _BN, _D), lambda i: (0, i, 0)),
            pl.BlockSpec((_NC, _BN, 1), lambda i: (0, i, 0)),
            pl.BlockSpec((_BN, _D), lambda i: (i, 0)),
            pl.BlockSpec((_BN, 1), lambda i: (i, 0)),
            pl.BlockSpec((_BN, 1), lambda i: (i, 0)),
            pl.BlockSpec((1, 1), lambda i: (0, 0)),
            pl.BlockSpec((1, _D), lambda i: (0, 0)),
        ],
        out_specs=pl.BlockSpec((_BN, _D), lambda i: (i, 0)),
        out_shape=jax.ShapeDtypeStruct((_NP, _D), jnp.float32),
    )(op, dp, xp, a_s, a_d, m, b)


def kernel(x, edge_index, edge_weight, W1, W_edge1, att_src1, att_dst1,
           att_edge1, bias1, W2, W_edge2, att_src2, att_dst2, att_edge2,
           bias2):
    xpad = jnp.pad(x, ((0, _NP - _N), (0, 0)))
    src1d = edge_index[0]
    dst1d = edge_index[1]
    ew2d = edge_weight.reshape(_E // _D, _D)
    ats1 = att_src1.reshape(1, _D)
    atd1 = att_dst1.reshape(1, _D)
    ate1 = att_edge1.reshape(1, _D)
    ats2 = att_src2.reshape(1, _D)
    atd2 = att_dst2.reshape(1, _D)
    ate2 = att_edge2.reshape(1, _D)
    b1 = bias1.reshape(1, _D)
    b2 = bias2.reshape(1, _D)
    zrow = jnp.zeros((_NP, _D), jnp.float32)
    zd = jnp.zeros((_NP,), jnp.float32)

    xp1, as1, ad1, ae1, ae2, m1, m2 = _tc_prep(
        xpad, W1, ats1, atd1, ew2d, W_edge1.reshape(1, _D), ate1,
        W_edge2.reshape(1, _D), ate2)
    op1, dp1 = _sc_edge()(src1d, dst1d, ae1.reshape(_E), xp1,
                          as1.reshape(_NP), ad1.reshape(_NP), zrow, zd)
    xp2, as2, ad2 = _tc_comb1(op1.reshape(_NC, _NP, _D),
                              dp1.reshape(_NC, _NP, 1),
                              xp1, as1, ad1, m1, b1, W2, ats2, atd2)
    op2, dp2 = _sc_edge()(src1d, dst1d, ae2.reshape(_E), xp2,
                          as2.reshape(_NP), ad2.reshape(_NP), zrow, zd)
    out = _tc_comb2(op2.reshape(_NC, _NP, _D), dp2.reshape(_NC, _NP, 1),
                    xp2, as2, ad2, m2, b2)
    return out[:_N]
